# baseline (speedup 1.0000x reference)
import functools

import jax
import jax.numpy as jnp
from jax import lax
from jax.experimental import pallas as pl
from jax.experimental.pallas import tpu as pltpu
from jax.experimental.pallas import tpu_sc as plsc

_N_CLASSES = 19
_NC, _NS, _L = 2, 16, 16
_NW = _NC * _NS
_BINS = _N_CLASSES * _L


def _sc_hist_body(t_hbm, out_hbm, chunk_v, bins_v):
    wid = lax.axis_index("s") * _NC + lax.axis_index("c")
    per_w = chunk_v.shape[0]
    base = wid * per_w

    zeros = jnp.zeros((_L,), jnp.float32)
    for i in range(_BINS // _L):
        bins_v[pl.ds(i * _L, _L)] = zeros

    pltpu.sync_copy(t_hbm.at[pl.ds(base, per_w)], chunk_v)

    lanes = lax.iota(jnp.int32, _L)
    ones = jnp.ones((_L,), jnp.float32)

    def step(i, carry):
        v = chunk_v[pl.ds(i * _L, _L)]
        plsc.addupdate_scatter(bins_v, [v * _L + lanes], ones)
        return carry

    lax.fori_loop(0, per_w // _L, step, 0)

    pltpu.sync_copy(bins_v, out_hbm.at[pl.ds(wid * _BINS, _BINS)])


def _sc_hist(tflat):
    per_w = tflat.shape[0] // _NW
    mesh = plsc.VectorSubcoreMesh(
        core_axis_name="c", subcore_axis_name="s",
        num_cores=_NC, num_subcores=_NS)
    return pl.kernel(
        _sc_hist_body,
        out_type=jax.ShapeDtypeStruct((_NW * _BINS,), jnp.float32),
        mesh=mesh,
        scratch_types=[
            pltpu.VMEM((per_w,), jnp.int32),
            pltpu.VMEM((_BINS,), jnp.float32),
        ],
        compiler_params=pltpu.CompilerParams(needs_layout_passes=False),
    )(tflat)


def _tc_body(x_ref, t_ref, acc_ref):
    @pl.when((pl.program_id(0) == 0) & (pl.program_id(1) == 0))
    def _init():
        acc_ref[...] = jnp.zeros_like(acc_ref)

    t = t_ref[0]
    c_dim = x_ref.shape[1]
    bh = t.shape[0]
    ones = jnp.ones((1, bh), dtype=jnp.float32)
    dot = functools.partial(
        jax.lax.dot_general,
        dimension_numbers=(((1,), (0,)), ((), ())),
        preferred_element_type=jnp.float32,
    )

    m = x_ref[0, 0]
    for c in range(1, c_dim):
        m = jnp.maximum(m, x_ref[0, c])

    s = jnp.zeros_like(m)
    for c in range(c_dim):
        xc = x_ref[0, c]
        s = s + jnp.exp(xc - m)
        eqf = (t == c).astype(jnp.float32)
        fnv = eqf * (xc < m).astype(jnp.float32)
        gxv = eqf * xc
        acc_ref[0, c] += dot(ones, fnv)[0]
        acc_ref[1, c] += dot(ones, gxv)[0]

    lse = m + jnp.log(s)
    for c in range(c_dim):
        eqf = (t == c).astype(jnp.float32)
        acc_ref[2, c] += dot(ones, eqf * lse)[0]


@jax.jit
def kernel(input, target):
    b_dim, c_dim, h_dim, w_dim = input.shape
    bh = 64
    gt_parts = _sc_hist(target.reshape(-1))
    acc = pl.pallas_call(
        _tc_body,
        grid=(b_dim, h_dim // bh),
        in_specs=[
            pl.BlockSpec((1, c_dim, bh, w_dim), lambda b, h: (b, 0, h, 0)),
            pl.BlockSpec((1, bh, w_dim), lambda b, h: (b, h, 0)),
        ],
        out_specs=pl.BlockSpec((3, c_dim, w_dim), lambda b, h: (0, 0, 0)),
        out_shape=jax.ShapeDtypeStruct((3, c_dim, w_dim), jnp.float32),
    )(input, target)
    gt = jnp.sum(gt_parts.reshape(_NW, _N_CLASSES, _L), axis=(0, 2))
    fn = jnp.sum(acc[0], axis=-1)
    gx = jnp.sum(acc[1], axis=-1)
    lse = jnp.sum(acc[2], axis=-1)
    ces = lse - gx
    weight = jnp.where(fn > 0, fn, 1.0) / jnp.where(gt > 0, gt, 1.0)
    return jnp.sum(weight * ces) / (b_dim * h_dim * w_dim)


# speedup vs baseline: 1.0016x; 1.0016x over previous
import functools

import jax
import jax.numpy as jnp
from jax import lax
from jax.experimental import pallas as pl
from jax.experimental.pallas import tpu as pltpu
from jax.experimental.pallas import tpu_sc as plsc

_N_CLASSES = 19
_NC, _NS, _L = 2, 16, 16
_NW = _NC * _NS
_BINS = _N_CLASSES * _L


def _sc_hist_body(t_hbm, out_hbm, chunk_v, bins_v):
    wid = lax.axis_index("s") * _NC + lax.axis_index("c")
    per_w = chunk_v.shape[0]
    base = wid * per_w

    zeros = jnp.zeros((_L,), jnp.float32)
    for i in range(_BINS // _L):
        bins_v[pl.ds(i * _L, _L)] = zeros

    pltpu.sync_copy(t_hbm.at[pl.ds(base, per_w)], chunk_v)

    lanes = lax.iota(jnp.int32, _L)
    ones = jnp.ones((_L,), jnp.float32)

    unroll = 16
    def step(i, carry):
        base2 = i * (_L * unroll)
        for u in range(unroll):
            v = chunk_v[pl.ds(base2 + u * _L, _L)]
            plsc.addupdate_scatter(bins_v, [v * _L + lanes], ones)
        return carry

    lax.fori_loop(0, per_w // (_L * unroll), step, 0)

    pltpu.sync_copy(bins_v, out_hbm.at[pl.ds(wid * _BINS, _BINS)])


def _sc_hist(tflat):
    per_w = tflat.shape[0] // _NW
    mesh = plsc.VectorSubcoreMesh(
        core_axis_name="c", subcore_axis_name="s",
        num_cores=_NC, num_subcores=_NS)
    return pl.kernel(
        _sc_hist_body,
        out_type=jax.ShapeDtypeStruct((_NW * _BINS,), jnp.float32),
        mesh=mesh,
        scratch_types=[
            pltpu.VMEM((per_w,), jnp.int32),
            pltpu.VMEM((_BINS,), jnp.float32),
        ],
        compiler_params=pltpu.CompilerParams(needs_layout_passes=False),
    )(tflat)


def _tc_body(x_ref, t_ref, acc_ref):
    @pl.when((pl.program_id(0) == 0) & (pl.program_id(1) == 0))
    def _init():
        acc_ref[...] = jnp.zeros_like(acc_ref)

    t = t_ref[0]
    c_dim = x_ref.shape[1]
    bh = t.shape[0]
    ones = jnp.ones((1, bh), dtype=jnp.float32)
    dot = functools.partial(
        jax.lax.dot_general,
        dimension_numbers=(((1,), (0,)), ((), ())),
        preferred_element_type=jnp.float32,
    )

    m = x_ref[0, 0]
    for c in range(1, c_dim):
        m = jnp.maximum(m, x_ref[0, c])

    s = jnp.zeros_like(m)
    for c in range(c_dim):
        xc = x_ref[0, c]
        s = s + jnp.exp(xc - m)
        eqf = (t == c).astype(jnp.float32)
        fnv = eqf * (xc < m).astype(jnp.float32)
        gxv = eqf * xc
        acc_ref[0, c] += dot(ones, fnv)[0]
        acc_ref[1, c] += dot(ones, gxv)[0]

    lse = m + jnp.log(s)
    for c in range(c_dim):
        eqf = (t == c).astype(jnp.float32)
        acc_ref[2, c] += dot(ones, eqf * lse)[0]


@jax.jit
def kernel(input, target):
    b_dim, c_dim, h_dim, w_dim = input.shape
    bh = 64
    gt_parts = _sc_hist(target.reshape(-1))
    acc = pl.pallas_call(
        _tc_body,
        grid=(b_dim, h_dim // bh),
        in_specs=[
            pl.BlockSpec((1, c_dim, bh, w_dim), lambda b, h: (b, 0, h, 0)),
            pl.BlockSpec((1, bh, w_dim), lambda b, h: (b, h, 0)),
        ],
        out_specs=pl.BlockSpec((3, c_dim, w_dim), lambda b, h: (0, 0, 0)),
        out_shape=jax.ShapeDtypeStruct((3, c_dim, w_dim), jnp.float32),
    )(input, target)
    gt = jnp.sum(gt_parts.reshape(_NW, _N_CLASSES, _L), axis=(0, 2))
    fn = jnp.sum(acc[0], axis=-1)
    gx = jnp.sum(acc[1], axis=-1)
    lse = jnp.sum(acc[2], axis=-1)
    ces = lse - gx
    weight = jnp.where(fn > 0, fn, 1.0) / jnp.where(gt > 0, gt, 1.0)
    return jnp.sum(weight * ces) / (b_dim * h_dim * w_dim)


# speedup vs baseline: 1.0877x; 1.0859x over previous
import functools

import jax
import jax.numpy as jnp
from jax import lax
from jax.experimental import pallas as pl
from jax.experimental.pallas import tpu as pltpu
from jax.experimental.pallas import tpu_sc as plsc

_N_CLASSES = 19
_NC, _NS, _L = 2, 16, 16
_NW = _NC * _NS
_BINS = _N_CLASSES * _L


def _sc_hist_body(t_hbm, out_hbm, chunk_v, bins_v):
    wid = lax.axis_index("s") * _NC + lax.axis_index("c")
    rows, cols = chunk_v.shape
    quarters = t_hbm.shape[1] // rows
    b = wid // quarters
    q = wid % quarters

    zeros = jnp.zeros((_L,), jnp.float32)
    for i in range(_BINS // _L):
        bins_v[pl.ds(i * _L, _L)] = zeros

    pltpu.sync_copy(t_hbm.at[b, pl.ds(q * rows, rows)], chunk_v)

    lanes = lax.iota(jnp.int32, _L)
    ones = jnp.ones((_L,), jnp.float32)

    def step(i, carry):
        for u in range(cols // _L):
            v = chunk_v[i, pl.ds(u * _L, _L)]
            plsc.addupdate_scatter(bins_v, [v * _L + lanes], ones)
        return carry

    lax.fori_loop(0, rows, step, 0)

    pltpu.sync_copy(bins_v, out_hbm.at[pl.ds(wid * _BINS, _BINS)])


def _sc_hist(target):
    b_dim, h_dim, w_dim = target.shape
    rows = (b_dim * h_dim) // _NW
    mesh = plsc.VectorSubcoreMesh(
        core_axis_name="c", subcore_axis_name="s",
        num_cores=_NC, num_subcores=_NS)
    return pl.kernel(
        _sc_hist_body,
        out_type=jax.ShapeDtypeStruct((_NW * _BINS,), jnp.float32),
        mesh=mesh,
        scratch_types=[
            pltpu.VMEM((rows, w_dim), jnp.int32),
            pltpu.VMEM((_BINS,), jnp.float32),
        ],
        compiler_params=pltpu.CompilerParams(needs_layout_passes=False),
    )(target)


def _tc_body(x_ref, t_ref, acc_ref):
    @pl.when((pl.program_id(0) == 0) & (pl.program_id(1) == 0))
    def _init():
        acc_ref[...] = jnp.zeros_like(acc_ref)

    t = t_ref[0]
    c_dim = x_ref.shape[1]
    bh = t.shape[0]
    ones = jnp.ones((1, bh), dtype=jnp.float32)
    dot = functools.partial(
        jax.lax.dot_general,
        dimension_numbers=(((1,), (0,)), ((), ())),
        preferred_element_type=jnp.float32,
    )

    m = x_ref[0, 0]
    for c in range(1, c_dim):
        m = jnp.maximum(m, x_ref[0, c])

    s = jnp.zeros_like(m)
    for c in range(c_dim):
        xc = x_ref[0, c]
        s = s + jnp.exp(xc - m)
        eqf = (t == c).astype(jnp.float32)
        fnv = eqf * (xc < m).astype(jnp.float32)
        gxv = eqf * xc
        acc_ref[0, c] += dot(ones, fnv)[0]
        acc_ref[1, c] += dot(ones, gxv)[0]

    lse = m + jnp.log(s)
    for c in range(c_dim):
        eqf = (t == c).astype(jnp.float32)
        acc_ref[2, c] += dot(ones, eqf * lse)[0]


@jax.jit
def kernel(input, target):
    b_dim, c_dim, h_dim, w_dim = input.shape
    bh = 64
    gt_parts = _sc_hist(target)
    acc = pl.pallas_call(
        _tc_body,
        grid=(b_dim, h_dim // bh),
        in_specs=[
            pl.BlockSpec((1, c_dim, bh, w_dim), lambda b, h: (b, 0, h, 0)),
            pl.BlockSpec((1, bh, w_dim), lambda b, h: (b, h, 0)),
        ],
        out_specs=pl.BlockSpec((3, c_dim, w_dim), lambda b, h: (0, 0, 0)),
        out_shape=jax.ShapeDtypeStruct((3, c_dim, w_dim), jnp.float32),
    )(input, target)
    gt = jnp.sum(gt_parts.reshape(_NW, _N_CLASSES, _L), axis=(0, 2))
    fn = jnp.sum(acc[0], axis=-1)
    gx = jnp.sum(acc[1], axis=-1)
    lse = jnp.sum(acc[2], axis=-1)
    ces = lse - gx
    weight = jnp.where(fn > 0, fn, 1.0) / jnp.where(gt > 0, gt, 1.0)
    return jnp.sum(weight * ces) / (b_dim * h_dim * w_dim)


# speedup vs baseline: 1.2396x; 1.1397x over previous
import functools

import jax
import jax.numpy as jnp
from jax import lax
from jax.experimental import pallas as pl
from jax.experimental.pallas import tpu as pltpu
from jax.experimental.pallas import tpu_sc as plsc

_N_CLASSES = 19
_NC, _NS, _L = 2, 16, 16
_NW = _NC * _NS
_BINS = _N_CLASSES * _L


def _sc_hist_body(t_hbm, out_hbm, chunk_v, bins_v):
    wid = lax.axis_index("s") * _NC + lax.axis_index("c")
    rows, cols = chunk_v.shape
    quarters = t_hbm.shape[1] // rows
    b = wid // quarters
    q = wid % quarters

    zeros = jnp.zeros((_L,), jnp.float32)
    for i in range(_BINS // _L):
        bins_v[pl.ds(i * _L, _L)] = zeros

    pltpu.sync_copy(t_hbm.at[b, pl.ds(q * rows, rows)], chunk_v)

    lanes = lax.iota(jnp.int32, _L)
    ones = jnp.ones((_L,), jnp.float32)

    def step(i, carry):
        for u in range(cols // _L):
            v = chunk_v[i, pl.ds(u * _L, _L)]
            plsc.addupdate_scatter(bins_v, [v * _L + lanes], ones)
        return carry

    lax.fori_loop(0, rows, step, 0)

    pltpu.sync_copy(bins_v, out_hbm.at[pl.ds(wid * _BINS, _BINS)])


def _sc_hist(target):
    b_dim, h_dim, w_dim = target.shape
    rows = (b_dim * h_dim) // _NW
    mesh = plsc.VectorSubcoreMesh(
        core_axis_name="c", subcore_axis_name="s",
        num_cores=_NC, num_subcores=_NS)
    return pl.kernel(
        _sc_hist_body,
        out_type=jax.ShapeDtypeStruct((_NW * _BINS,), jnp.float32),
        mesh=mesh,
        scratch_types=[
            pltpu.VMEM((rows, w_dim), jnp.int32),
            pltpu.VMEM((_BINS,), jnp.float32),
        ],
        compiler_params=pltpu.CompilerParams(needs_layout_passes=False),
    )(target)


def _tc_body(x_ref, t_ref, acc_ref):
    @pl.when((pl.program_id(0) == 0) & (pl.program_id(1) == 0))
    def _init():
        acc_ref[...] = jnp.zeros_like(acc_ref)

    t = t_ref[0]
    c_dim = x_ref.shape[1]
    bh = t.shape[0]
    ones = jnp.ones((1, bh), dtype=jnp.float32)
    dot = functools.partial(
        jax.lax.dot_general,
        dimension_numbers=(((1,), (0,)), ((), ())),
        preferred_element_type=jnp.float32,
    )

    m = x_ref[0, 0]
    for c in range(1, c_dim):
        m = jnp.maximum(m, x_ref[0, c])

    s = jnp.zeros_like(m)
    for c in range(c_dim):
        xc = x_ref[0, c]
        s = s + jnp.exp(xc - m)
        eqf = (t == c).astype(jnp.float32)
        fnv = eqf * (xc < m).astype(jnp.float32)
        gxv = eqf * xc
        acc_ref[0, c] += dot(ones, fnv)[0]
        acc_ref[1, c] += dot(ones, gxv)[0]

    lse = m + jnp.log(s)
    for c in range(c_dim):
        eqf = (t == c).astype(jnp.float32)
        acc_ref[2, c] += dot(ones, eqf * lse)[0]


@jax.jit
def kernel(input, target):
    b_dim, c_dim, h_dim, w_dim = input.shape
    bh = 128
    gt_parts = _sc_hist(target)
    acc = pl.pallas_call(
        _tc_body,
        grid=(b_dim, h_dim // bh),
        in_specs=[
            pl.BlockSpec((1, c_dim, bh, w_dim), lambda b, h: (b, 0, h, 0)),
            pl.BlockSpec((1, bh, w_dim), lambda b, h: (b, h, 0)),
        ],
        out_specs=pl.BlockSpec((3, c_dim, w_dim), lambda b, h: (0, 0, 0)),
        out_shape=jax.ShapeDtypeStruct((3, c_dim, w_dim), jnp.float32),
    )(input, target)
    gt = jnp.sum(gt_parts.reshape(_NW, _N_CLASSES, _L), axis=(0, 2))
    fn = jnp.sum(acc[0], axis=-1)
    gx = jnp.sum(acc[1], axis=-1)
    lse = jnp.sum(acc[2], axis=-1)
    ces = lse - gx
    weight = jnp.where(fn > 0, fn, 1.0) / jnp.where(gt > 0, gt, 1.0)
    return jnp.sum(weight * ces) / (b_dim * h_dim * w_dim)


# speedup vs baseline: 1.2466x; 1.0056x over previous
import functools

import jax
import jax.numpy as jnp
from jax import lax
from jax.experimental import pallas as pl
from jax.experimental.pallas import tpu as pltpu
from jax.experimental.pallas import tpu_sc as plsc

_N_CLASSES = 19
_NC, _NS, _L = 2, 16, 16
_NW = _NC * _NS
_BINS = _N_CLASSES * _L


def _sc_hist_body(t_hbm, out_hbm, chunk_v, bins_v):
    wid = lax.axis_index("s") * _NC + lax.axis_index("c")
    rows, cols = chunk_v.shape
    quarters = t_hbm.shape[1] // rows
    b = wid // quarters
    q = wid % quarters

    zeros = jnp.zeros((_L,), jnp.float32)
    for i in range(_BINS // _L):
        bins_v[pl.ds(i * _L, _L)] = zeros

    pltpu.sync_copy(t_hbm.at[b, pl.ds(q * rows, rows)], chunk_v)

    lanes = lax.iota(jnp.int32, _L)
    ones = jnp.ones((_L,), jnp.float32)

    def step(i, carry):
        for u in range(cols // _L):
            v = chunk_v[i, pl.ds(u * _L, _L)]
            plsc.addupdate_scatter(bins_v, [v * _L + lanes], ones)
        return carry

    lax.fori_loop(0, rows, step, 0)

    pltpu.sync_copy(bins_v, out_hbm.at[pl.ds(wid * _BINS, _BINS)])


def _sc_hist(target):
    b_dim, h_dim, w_dim = target.shape
    rows = (b_dim * h_dim) // _NW
    mesh = plsc.VectorSubcoreMesh(
        core_axis_name="c", subcore_axis_name="s",
        num_cores=_NC, num_subcores=_NS)
    return pl.kernel(
        _sc_hist_body,
        out_type=jax.ShapeDtypeStruct((_NW * _BINS,), jnp.float32),
        mesh=mesh,
        scratch_types=[
            pltpu.VMEM((rows, w_dim), jnp.int32),
            pltpu.VMEM((_BINS,), jnp.float32),
        ],
        compiler_params=pltpu.CompilerParams(needs_layout_passes=False),
    )(target)


def _tc_body(x_ref, t_ref, acc_ref):
    @pl.when((pl.program_id(0) == 0) & (pl.program_id(1) == 0))
    def _init():
        acc_ref[...] = jnp.zeros_like(acc_ref)

    t = t_ref[0]
    c_dim = x_ref.shape[1]
    bh = t.shape[0]
    ones = jnp.ones((1, bh), dtype=jnp.float32)
    dot = functools.partial(
        jax.lax.dot_general,
        dimension_numbers=(((1,), (0,)), ((), ())),
        preferred_element_type=jnp.float32,
    )

    m = x_ref[0, 0]
    for c in range(1, c_dim):
        m = jnp.maximum(m, x_ref[0, c])

    s = jnp.zeros_like(m)
    for c in range(c_dim):
        xc = x_ref[0, c]
        s = s + jnp.exp(xc - m)
        eqf = (t == c).astype(jnp.float32)
        fnv = eqf * (xc < m).astype(jnp.float32)
        gxv = eqf * xc
        acc_ref[0, c] += dot(ones, fnv)[0]
        acc_ref[1, c] += dot(ones, gxv)[0]

    lse = m + jnp.log(s)
    for c in range(c_dim):
        eqf = (t == c).astype(jnp.float32)
        acc_ref[2, c] += dot(ones, eqf * lse)[0]


@jax.jit
def kernel(input, target):
    b_dim, c_dim, h_dim, w_dim = input.shape
    bh = 256
    gt_parts = _sc_hist(target)
    acc = pl.pallas_call(
        _tc_body,
        grid=(b_dim, h_dim // bh),
        in_specs=[
            pl.BlockSpec((1, c_dim, bh, w_dim), lambda b, h: (b, 0, h, 0)),
            pl.BlockSpec((1, bh, w_dim), lambda b, h: (b, h, 0)),
        ],
        out_specs=pl.BlockSpec((3, c_dim, w_dim), lambda b, h: (0, 0, 0)),
        out_shape=jax.ShapeDtypeStruct((3, c_dim, w_dim), jnp.float32),
    )(input, target)
    gt = jnp.sum(gt_parts.reshape(_NW, _N_CLASSES, _L), axis=(0, 2))
    fn = jnp.sum(acc[0], axis=-1)
    gx = jnp.sum(acc[1], axis=-1)
    lse = jnp.sum(acc[2], axis=-1)
    ces = lse - gx
    weight = jnp.where(fn > 0, fn, 1.0) / jnp.where(gt > 0, gt, 1.0)
    return jnp.sum(weight * ces) / (b_dim * h_dim * w_dim)
